# baseline (speedup 1.0000x reference)
import functools

import jax
import jax.numpy as jnp
from jax import lax
from jax.experimental import pallas as pl
from jax.experimental.pallas import tpu as pltpu
from jax.experimental.pallas import tpu_sc as plsc

_NC = 2
_NS = 16
_NW = _NC * _NS
_LANES = 16

_LOG1P_C = (0.99745014, -0.47131087, 0.22570627, -0.05876987)


def _proj_body(w_ref, b_ref, x_ref, p_ref):
    h = x_ref.shape[1]
    wd = w_ref[1:2, :] - w_ref[0:1, :]
    lhs = jnp.concatenate([wd[:, :h], wd[:, h:]], axis=0)
    p = lax.dot_general(lhs, x_ref[...], (((1,), (1,)), ((), ())),
                        preferred_element_type=jnp.float32)
    row = lax.broadcasted_iota(jnp.int32, p.shape, 0)
    p_ref[...] = p + jnp.where(row == 0, b_ref[1] - b_ref[0], 0.0)


def _softplus(d):
    t = jnp.exp(-jnp.abs(d))
    acc = jnp.full(d.shape, _LOG1P_C[-1], jnp.float32)
    for c in _LOG1P_C[-2::-1]:
        acc = acc * t + c
    return jnp.maximum(d, 0.0) + acc * t


def _make_sc_kernel(n_nodes, n_edges, cols, cols_last):
    mesh = plsc.VectorSubcoreMesh(core_axis_name="c", subcore_axis_name="s")
    nb = cols // 128
    nb_last = cols_last // 128

    @functools.partial(
        pl.kernel,
        out_type=jax.ShapeDtypeStruct((n_edges // 128, 2, 128), jnp.float32),
        mesh=mesh,
        scratch_types=[
            pltpu.VMEM((2, n_nodes), jnp.float32),
            pltpu.VMEM((2, cols_last), jnp.int32),
            pltpu.VMEM((cols_last // 128, 2, 128), jnp.float32),
            pltpu.SemaphoreType.DMA,
            pltpu.SemaphoreType.DMA,
        ],
        compiler_params=pltpu.CompilerParams(needs_layout_passes=False),
    )
    def sc_kernel(t_hbm, edges_hbm, out_hbm,
                  t_v, e_v, ov_v, sem_t, sem_e):
        wid = lax.axis_index("s") * _NC + lax.axis_index("c")
        base = wid * cols
        cp_t = pltpu.async_copy(t_hbm, t_v, sem_t)
        cp_e = pltpu.async_copy(edges_hbm.at[:, pl.ds(base, cols_last)],
                                e_v, sem_e)
        cp_t.wait()
        cp_e.wait()
        zero16 = jnp.zeros((_LANES,), jnp.int32)
        one16 = zero16 + 1
        n_blocks = jnp.where(wid == _NW - 1, nb_last, nb)

        @plsc.parallel_loop(0, n_blocks, 1)
        def _(g):
            for k in range(8):
                off = g * 128 + k * _LANES
                idx_s = e_v[0, pl.ds(off, _LANES)]
                idx_d = e_v[1, pl.ds(off, _LANES)]
                a = plsc.load_gather(t_v, [zero16, idx_s])
                c = plsc.load_gather(t_v, [one16, idx_d])
                d = a + c
                sp = _softplus(d)
                ov_v[g, 0, pl.ds(k * _LANES, _LANES)] = -sp
                ov_v[g, 1, pl.ds(k * _LANES, _LANES)] = d - sp

        gbase = wid * nb
        pltpu.sync_copy(ov_v.at[pl.ds(0, nb)], out_hbm.at[pl.ds(gbase, nb)])

        @pl.when(wid == _NW - 1)
        def _():
            pltpu.sync_copy(ov_v.at[pl.ds(nb, nb_last - nb)],
                            out_hbm.at[pl.ds(gbase + nb, nb_last - nb)])

    return sc_kernel


def kernel(node_features_after_gcn, edges, W, b):
    x = node_features_after_gcn
    n_nodes, hidden = x.shape
    n_edges = edges.shape[1]

    cols = (n_edges // _NW) // 128 * 128
    cols_last = n_edges - (_NW - 1) * cols

    rows = 2560
    proj = pl.pallas_call(
        _proj_body,
        grid=(4,),
        in_specs=[
            pl.BlockSpec((2, 2 * hidden), lambda i: (0, 0)),
            pl.BlockSpec(memory_space=pltpu.SMEM),
            pl.BlockSpec((rows, hidden), lambda i: (i, 0)),
        ],
        out_specs=pl.BlockSpec((2, rows), lambda i: (0, i)),
        out_shape=jax.ShapeDtypeStruct((2, n_nodes), jnp.float32),
    )(W, b, x)

    flat = _make_sc_kernel(n_nodes, n_edges, cols, cols_last)(proj, edges)

    return flat.transpose(0, 2, 1).reshape(n_edges, 2)


# speedup vs baseline: 1.0266x; 1.0266x over previous
import functools

import jax
import jax.numpy as jnp
from jax import lax
from jax.experimental import pallas as pl
from jax.experimental.pallas import tpu as pltpu
from jax.experimental.pallas import tpu_sc as plsc

_NC = 2
_NS = 16
_NW = _NC * _NS
_LANES = 16

_LOG1P_C = (0.98746072, -0.40843993, 0.11466497)


def _proj_body(w_ref, b_ref, x_ref, p_ref):
    h = x_ref.shape[1]
    wd = w_ref[1:2, :] - w_ref[0:1, :]
    lhs = jnp.concatenate([wd[:, :h], wd[:, h:]], axis=0)
    p = lax.dot_general(lhs, x_ref[...], (((1,), (1,)), ((), ())),
                        preferred_element_type=jnp.float32)
    row = lax.broadcasted_iota(jnp.int32, p.shape, 0)
    p_ref[...] = p + jnp.where(row == 0, b_ref[1] - b_ref[0], 0.0)


def _softplus(d):
    t = jnp.exp(-jnp.abs(d))
    acc = jnp.full(d.shape, _LOG1P_C[-1], jnp.float32)
    for c in _LOG1P_C[-2::-1]:
        acc = acc * t + c
    return jnp.maximum(d, 0.0) + acc * t


def _make_sc_kernel(n_nodes, n_edges, cols, cols_last):
    mesh = plsc.VectorSubcoreMesh(core_axis_name="c", subcore_axis_name="s")
    nb = cols // 128
    nb_last = cols_last // 128

    @functools.partial(
        pl.kernel,
        out_type=jax.ShapeDtypeStruct((n_edges // 128, 2, 128), jnp.float32),
        mesh=mesh,
        scratch_types=[
            pltpu.VMEM((2, n_nodes), jnp.float32),
            pltpu.VMEM((2, cols_last), jnp.int32),
            pltpu.VMEM((cols_last // 128, 2, 128), jnp.float32),
            pltpu.SemaphoreType.DMA,
            pltpu.SemaphoreType.DMA,
        ],
        compiler_params=pltpu.CompilerParams(needs_layout_passes=False),
    )
    def sc_kernel(t_hbm, edges_hbm, out_hbm,
                  t_v, e_v, ov_v, sem_t, sem_e):
        wid = lax.axis_index("s") * _NC + lax.axis_index("c")
        base = wid * cols
        cp_t = pltpu.async_copy(t_hbm, t_v, sem_t)
        cp_e = pltpu.async_copy(edges_hbm.at[:, pl.ds(base, cols_last)],
                                e_v, sem_e)
        cp_t.wait()
        cp_e.wait()
        zero16 = jnp.zeros((_LANES,), jnp.int32)
        one16 = zero16 + 1
        n_blocks = jnp.where(wid == _NW - 1, nb_last, nb)

        @plsc.parallel_loop(0, n_blocks, 1)
        def _(g):
            for k in range(8):
                off = g * 128 + k * _LANES
                idx_s = e_v[0, pl.ds(off, _LANES)]
                idx_d = e_v[1, pl.ds(off, _LANES)]
                a = plsc.load_gather(t_v, [zero16, idx_s])
                c = plsc.load_gather(t_v, [one16, idx_d])
                d = a + c
                sp = _softplus(d)
                ov_v[g, 0, pl.ds(k * _LANES, _LANES)] = -sp
                ov_v[g, 1, pl.ds(k * _LANES, _LANES)] = d - sp

        gbase = wid * nb
        pltpu.sync_copy(ov_v.at[pl.ds(0, nb)], out_hbm.at[pl.ds(gbase, nb)])

        @pl.when(wid == _NW - 1)
        def _():
            pltpu.sync_copy(ov_v.at[pl.ds(nb, nb_last - nb)],
                            out_hbm.at[pl.ds(gbase + nb, nb_last - nb)])

    return sc_kernel


def kernel(node_features_after_gcn, edges, W, b):
    x = node_features_after_gcn
    n_nodes, hidden = x.shape
    n_edges = edges.shape[1]

    cols = (n_edges // _NW) // 128 * 128
    cols_last = n_edges - (_NW - 1) * cols

    rows = 5120
    proj = pl.pallas_call(
        _proj_body,
        grid=(2,),
        in_specs=[
            pl.BlockSpec((2, 2 * hidden), lambda i: (0, 0)),
            pl.BlockSpec(memory_space=pltpu.SMEM),
            pl.BlockSpec((rows, hidden), lambda i: (i, 0)),
        ],
        out_specs=pl.BlockSpec((2, rows), lambda i: (0, i)),
        out_shape=jax.ShapeDtypeStruct((2, n_nodes), jnp.float32),
    )(W, b, x)

    flat = _make_sc_kernel(n_nodes, n_edges, cols, cols_last)(proj, edges)

    return flat.transpose(0, 2, 1).reshape(n_edges, 2)
